# baseline (speedup 1.0000x reference)
import functools

import jax
import jax.numpy as jnp
from jax import lax
from jax.experimental import pallas as pl
from jax.experimental.pallas import tpu as pltpu
from jax.experimental.pallas import tpu_sc as plsc

_CONTEXT = 20
_VOCAB = 100000
_EMBED = 64
_HIDDEN = 100

_NW = 32
_SPAN = 3200
_CH = 512
_WIDTHS = (512, 512, 512, 512, 512, 512, 128)
_OFFS = (0, 512, 1024, 1536, 2048, 2560, 3072)
_TAIL0 = 31 * _SPAN
_TAIL1 = _TAIL0 + _CH
_CHT = _VOCAB - _TAIL1
_NEG = -1e30


def _hidden_body(idx_ref, emb_hbm, w1_ref, b1_ref, h_ref, emb_vmem, dma_sem):
    for i in range(_CONTEXT):
        pltpu.make_async_copy(
            emb_hbm.at[pl.ds(idx_ref[i], 1), :],
            emb_vmem.at[pl.ds(i, 1), :],
            dma_sem,
        ).start()
    for i in range(_CONTEXT):
        pltpu.make_async_copy(
            emb_hbm.at[pl.ds(idx_ref[i], 1), :],
            emb_vmem.at[pl.ds(i, 1), :],
            dma_sem,
        ).wait()
    acc = b1_ref[...]
    for i in range(_CONTEXT):
        acc = acc + jnp.dot(
            emb_vmem[pl.ds(i, 1), :],
            w1_ref[pl.ds(i * _EMBED, _EMBED), :],
            preferred_element_type=jnp.float32,
        )
    h_ref[...] = jnp.tanh(acc)


def _hidden(inputs, emb_table, W1, b1):
    return pl.pallas_call(
        _hidden_body,
        grid_spec=pltpu.PrefetchScalarGridSpec(
            num_scalar_prefetch=1,
            grid=(1,),
            in_specs=[
                pl.BlockSpec(memory_space=pl.ANY),
                pl.BlockSpec((_CONTEXT * _EMBED, _HIDDEN), lambda i, idx: (0, 0)),
                pl.BlockSpec((1, _HIDDEN), lambda i, idx: (0, 0)),
            ],
            out_specs=pl.BlockSpec((1, _HIDDEN), lambda i, idx: (0, 0)),
            scratch_shapes=[
                pltpu.VMEM((_CONTEXT, _EMBED), jnp.float32),
                pltpu.SemaphoreType.DMA,
            ],
        ),
        out_shape=jax.ShapeDtypeStruct((1, _HIDDEN), jnp.float32),
    )(inputs.astype(jnp.int32), emb_table, W1, b1.reshape(1, _HIDDEN))


def _sc_body(
    h_hbm,
    w2_hbm,
    b2_hbm,
    logits_hbm,
    m_hbm,
    s_hbm,
    h_vmem,
    b2_vmem,
    logits_vmem,
    stat_vmem,
    w2_a,
    w2_b,
    sem_a,
    sem_b,
):
    wid = lax.axis_index("s") * 2 + lax.axis_index("c")
    pltpu.sync_copy(h_hbm, h_vmem)
    bufs = (w2_a, w2_b)
    sems = (sem_a, sem_b)

    def fma_chunk(w2_vmem, width):
        ng = width // 16

        def fma_body(k, accs):
            hv = h_vmem[pl.ds(k * 16, 16)]
            return tuple(
                accs[g] + hv * w2_vmem[k, pl.ds(g * 16, 16)] for g in range(ng)
            )

        init = tuple(b2_vmem[pl.ds(g * 16, 16)] for g in range(ng))
        return lax.fori_loop(0, _HIDDEN, fma_body, init)

    def stats_update(m_vec, s_vec, accs):
        chunk_m = accs[0]
        for a in accs[1:]:
            chunk_m = jnp.maximum(chunk_m, a)
        m_new = jnp.maximum(m_vec, chunk_m)
        s_new = s_vec * jnp.exp(m_vec - m_new)
        for a in accs:
            s_new = s_new + jnp.exp(a - m_new)
        return m_new, s_new

    def write_stats(m_vec, s_vec):
        stat_vmem[pl.ds(0, 16)] = m_vec
        pltpu.sync_copy(
            stat_vmem.at[pl.ds(0, 16)], m_hbm.at[pl.ds(wid * 16, 16)]
        )
        stat_vmem[pl.ds(16, 16)] = s_vec
        pltpu.sync_copy(
            stat_vmem.at[pl.ds(16, 16)], s_hbm.at[pl.ds(wid * 16, 16)]
        )

    @pl.when(wid < _NW - 1)
    def _main():
        base = wid * _SPAN
        nch = len(_WIDTHS)
        pending = [
            pltpu.async_copy(
                w2_hbm.at[:, pl.ds(base, _WIDTHS[0])],
                bufs[0].at[:, pl.ds(0, _WIDTHS[0])],
                sems[0],
            )
        ]
        m_vec = jnp.full((16,), _NEG, jnp.float32)
        s_vec = jnp.zeros((16,), jnp.float32)
        for c in range(nch):
            w = _WIDTHS[c]
            off = base + _OFFS[c]
            if c + 1 < nch:
                pending.append(
                    pltpu.async_copy(
                        w2_hbm.at[:, pl.ds(base + _OFFS[c + 1], _WIDTHS[c + 1])],
                        bufs[(c + 1) % 2].at[:, pl.ds(0, _WIDTHS[c + 1])],
                        sems[(c + 1) % 2],
                    )
                )
            pltpu.sync_copy(b2_hbm.at[pl.ds(off, w)], b2_vmem.at[pl.ds(0, w)])
            pending.pop(0).wait()
            accs = fma_chunk(bufs[c % 2], w)
            for g in range(w // 16):
                logits_vmem[pl.ds(g * 16, 16)] = accs[g]
            pltpu.sync_copy(
                logits_vmem.at[pl.ds(0, w)], logits_hbm.at[pl.ds(off, w)]
            )
            m_vec, s_vec = stats_update(m_vec, s_vec, accs)
        write_stats(m_vec, s_vec)

    @pl.when(wid == _NW - 1)
    def _tail():
        write_stats(
            jnp.full((16,), _NEG, jnp.float32), jnp.zeros((16,), jnp.float32)
        )


def _sc_logits(h, W2, b2):
    mesh = plsc.VectorSubcoreMesh(core_axis_name="c", subcore_axis_name="s")
    run = pl.kernel(
        _sc_body,
        mesh=mesh,
        out_type=[
            jax.ShapeDtypeStruct((_VOCAB,), jnp.float32),
            jax.ShapeDtypeStruct((_NW * 16,), jnp.float32),
            jax.ShapeDtypeStruct((_NW * 16,), jnp.float32),
        ],
        scratch_types=[
            pltpu.VMEM((_HIDDEN * 16,), jnp.float32),
            pltpu.VMEM((_CH,), jnp.float32),
            pltpu.VMEM((_CH,), jnp.float32),
            pltpu.VMEM((32,), jnp.float32),
            pltpu.VMEM((_HIDDEN, _CH), jnp.float32),
            pltpu.VMEM((_HIDDEN, _CH), jnp.float32),
            pltpu.SemaphoreType.DMA,
            pltpu.SemaphoreType.DMA,
        ],
    )
    hb = jnp.broadcast_to(h.reshape(_HIDDEN, 1), (_HIDDEN, 16)).reshape(
        _HIDDEN * 16
    )
    return run(hb, W2, b2)


_NB2 = 13
_VB2 = 8192
_TAIL_BLK = 12
_TAIL_LANE = _TAIL0 - _TAIL_BLK * _VB2


def _norm_body(logits_ref, m_ref, s_ref, h_ref, w2t_ref, b2t_ref, out_ref):
    j = pl.program_id(0)
    xt = (
        jnp.dot(h_ref[...], w2t_ref[...], preferred_element_type=jnp.float32)
        + b2t_ref[...]
    )
    colt = _TAIL0 + jax.lax.broadcasted_iota(jnp.int32, (1, _SPAN), 1)
    xt_m = jnp.where(colt < _VOCAB, xt, -jnp.inf)
    m_sc = jnp.max(m_ref[...])
    m_g = jnp.maximum(m_sc, jnp.max(xt_m))
    s_g = jnp.sum(s_ref[...] * jnp.exp(m_ref[...] - m_g)) + jnp.sum(
        jnp.exp(xt_m - m_g)
    )
    lse = m_g + jnp.log(s_g)
    out_ref[...] = logits_ref[...] - lse

    @pl.when(j == _TAIL_BLK)
    def _():
        padded = jnp.concatenate(
            [
                jnp.zeros((1, _TAIL_LANE), jnp.float32),
                xt_m,
                jnp.zeros((1, _VB2 - _TAIL_LANE - _SPAN), jnp.float32),
            ],
            axis=1,
        )
        col = _TAIL_BLK * _VB2 + jax.lax.broadcasted_iota(
            jnp.int32, (1, _VB2), 1
        )
        out_ref[...] = jnp.where(
            col >= _TAIL0, padded - lse, logits_ref[...] - lse
        )


def _normalize(logits2d, m_part, s_part, h, W2, b2_2d):
    return pl.pallas_call(
        _norm_body,
        grid=(_NB2,),
        in_specs=[
            pl.BlockSpec((1, _VB2), lambda j: (0, j)),
            pl.BlockSpec((_NW, 16), lambda j: (0, 0)),
            pl.BlockSpec((_NW, 16), lambda j: (0, 0)),
            pl.BlockSpec((1, _HIDDEN), lambda j: (0, 0)),
            pl.BlockSpec((_HIDDEN, _SPAN), lambda j: (0, _NW - 1)),
            pl.BlockSpec((1, _SPAN), lambda j: (0, _NW - 1)),
        ],
        out_specs=pl.BlockSpec((1, _VB2), lambda j: (0, j)),
        out_shape=jax.ShapeDtypeStruct((1, _VOCAB), jnp.float32),
    )(logits2d, m_part, s_part, h, W2, b2_2d)


def kernel(inputs, emb_table, W1, b1, W2, b2):
    h = _hidden(inputs, emb_table, W1, b1)
    logits, m_part, s_part = _sc_logits(h, W2, b2)
    return _normalize(
        logits.reshape(1, _VOCAB),
        m_part.reshape(_NW, 16),
        s_part.reshape(_NW, 16),
        h,
        W2,
        b2.reshape(1, _VOCAB),
    )
